# Initial kernel scaffold; baseline (speedup 1.0000x reference)
import jax
import jax.numpy as jnp
from jax.experimental import pallas as pl


def kernel(A, h, gt0_w1, gt0_w2, gt1_w1, gcn_w, gcn_b, lin1_w, lin1_b, lin2_w, lin2_b):
    raise NotImplementedError("write your pallas kernel here")


# speedup vs baseline: 2.0042x; 2.0042x over previous
import jax
import jax.numpy as jnp
from jax import lax
from jax.experimental import pallas as pl
from jax.experimental.pallas import tpu as pltpu


def _mix_kernel(filt_ref, a_ref, out_ref):
    F = out_ref.shape[0]
    tm, N = a_ref.shape[1], a_ref.shape[2]
    t = pl.program_id(0)
    rows = lax.broadcasted_iota(jnp.int32, (tm, N), 0) + t * tm
    cols = lax.broadcasted_iota(jnp.int32, (tm, N), 1)
    diag = (rows == cols).astype(jnp.float32)
    for f in range(F):
        acc = filt_ref[f, 0] * a_ref[0]
        for e in range(1, 4):
            acc = acc + filt_ref[f, e] * a_ref[e]
        acc = acc + filt_ref[f, 4] * diag
        out_ref[f] = acc.astype(jnp.bfloat16)


def _mix(filt, A4, tm=256):
    _, N, _ = A4.shape
    tm = min(tm, N)
    F = filt.shape[0]
    return pl.pallas_call(
        _mix_kernel,
        out_shape=jax.ShapeDtypeStruct((F, N, N), jnp.bfloat16),
        grid=(N // tm,),
        in_specs=[
            pl.BlockSpec(memory_space=pltpu.MemorySpace.SMEM),
            pl.BlockSpec((4, tm, N), lambda t: (0, t, 0)),
        ],
        out_specs=pl.BlockSpec((F, tm, N), lambda t: (0, t, 0)),
        compiler_params=pltpu.CompilerParams(dimension_semantics=("parallel",)),
    )(filt, A4)


def _mmnorm_kernel(a_ref, b_ref, o_ref, acc_ref):
    k = pl.program_id(1)

    @pl.when(k == 0)
    def _init():
        acc_ref[...] = jnp.zeros_like(acc_ref)

    acc_ref[...] += jnp.dot(a_ref[0], b_ref[0],
                            preferred_element_type=jnp.float32)

    @pl.when(k == pl.num_programs(1) - 1)
    def _finalize():
        m = acc_ref[...]
        deg = jnp.sum(m, axis=0, keepdims=True)
        inv = jnp.where(deg == 0.0, 0.0, 1.0 / deg)
        o_ref[0] = (m * inv).astype(jnp.bfloat16)


def _mmnorm(A, B, tk=512):
    C, N, _ = A.shape
    tk = min(tk, N)
    return pl.pallas_call(
        _mmnorm_kernel,
        out_shape=jax.ShapeDtypeStruct((C, N, N), jnp.bfloat16),
        grid=(C, N // tk),
        in_specs=[
            pl.BlockSpec((1, N, tk), lambda c, k: (c, 0, k)),
            pl.BlockSpec((1, tk, N), lambda c, k: (c, k, 0)),
        ],
        out_specs=pl.BlockSpec((1, N, N), lambda c, k: (c, 0, 0)),
        scratch_shapes=[pltpu.VMEM((N, N), jnp.float32)],
        compiler_params=pltpu.CompilerParams(
            dimension_semantics=("parallel", "arbitrary")),
    )(A, B)


def _gcn_mlp_kernel(hn_ref, feat_ref, gw_ref, gb_ref,
                    w1_ref, b1_ref, w2_ref, b2_ref, y_ref, h1_acc):
    c = pl.program_id(0)

    @pl.when(c == 0)
    def _init():
        h1_acc[...] = jnp.zeros_like(h1_acc)

    m = hn_ref[0]
    nz = (m != 0).astype(jnp.float32)
    out_deg = jnp.maximum(jnp.sum(nz, axis=1, keepdims=True), 1.0)
    in_deg = jnp.maximum(jnp.sum(nz, axis=0, keepdims=True), 1.0)

    feat = (feat_ref[...] * lax.rsqrt(out_deg)).astype(jnp.bfloat16)
    agg = lax.dot_general(m, feat,
                          dimension_numbers=(((0,), (0,)), ((), ())),
                          preferred_element_type=jnp.float32)
    agg = agg * lax.rsqrt(in_deg).T
    x_c = jnp.maximum(
        jnp.dot(agg.astype(jnp.bfloat16), gw_ref[...],
                preferred_element_type=jnp.float32) + gb_ref[...],
        0.0)

    h1_acc[...] += jnp.dot(x_c.astype(jnp.bfloat16), w1_ref[...],
                           preferred_element_type=jnp.float32)

    @pl.when(c == pl.num_programs(0) - 1)
    def _finalize():
        h1 = jnp.maximum(h1_acc[...] + b1_ref[...], 0.0)
        y_ref[...] = jnp.dot(h1.astype(jnp.bfloat16), w2_ref[...],
                             preferred_element_type=jnp.float32) + b2_ref[...]


def _gcn_mlp(Hn, feat, gcn_w, gcn_b, lin1_w, lin1_b, lin2_w, lin2_b):
    C, N, _ = Hn.shape
    w_in, w_out = gcn_w.shape
    lin1_out = lin1_w.shape[1]
    num_class = lin2_w.shape[1]
    return pl.pallas_call(
        _gcn_mlp_kernel,
        out_shape=jax.ShapeDtypeStruct((N, num_class), jnp.float32),
        grid=(C,),
        in_specs=[
            pl.BlockSpec((1, N, N), lambda c: (c, 0, 0)),
            pl.BlockSpec((N, w_in), lambda c: (0, 0)),
            pl.BlockSpec((w_in, w_out), lambda c: (0, 0)),
            pl.BlockSpec((1, w_out), lambda c: (0, 0)),
            pl.BlockSpec((w_out, lin1_out), lambda c: (c, 0)),
            pl.BlockSpec((1, lin1_out), lambda c: (0, 0)),
            pl.BlockSpec((lin1_out, num_class), lambda c: (0, 0)),
            pl.BlockSpec((1, num_class), lambda c: (0, 0)),
        ],
        out_specs=pl.BlockSpec((N, num_class), lambda c: (0, 0)),
        scratch_shapes=[pltpu.VMEM((N, lin1_out), jnp.float32)],
        compiler_params=pltpu.CompilerParams(dimension_semantics=("arbitrary",)),
    )(Hn, feat, gcn_w.astype(jnp.bfloat16), gcn_b,
      lin1_w.astype(jnp.bfloat16), lin1_b, lin2_w.astype(jnp.bfloat16), lin2_b)


def kernel(A, h, gt0_w1, gt0_w2, gt1_w1,
           gcn_w, gcn_b, lin1_w, lin1_b, lin2_w, lin2_b):
    C = gt0_w1.shape[0]
    f1 = jax.nn.softmax(gt0_w1, axis=1)
    f2 = jax.nn.softmax(gt0_w2, axis=1)
    fl1 = jax.nn.softmax(gt1_w1, axis=1)
    filt = jnp.concatenate([f1, f2, fl1], axis=0)

    M6 = _mix(filt, A[:4])

    H = _mmnorm(M6[:C], M6[C:2 * C])
    H = _mmnorm(H, M6[2 * C:])

    return _gcn_mlp(H, h, gcn_w, gcn_b, lin1_w, lin1_b, lin2_w, lin2_b)


# speedup vs baseline: 2.0102x; 1.0030x over previous
import jax
import jax.numpy as jnp
from jax import lax
from jax.experimental import pallas as pl
from jax.experimental.pallas import tpu as pltpu


def _mix_kernel(filt_ref, a_ref, out_ref):
    F = out_ref.shape[0]
    tm, N = a_ref.shape[1], a_ref.shape[2]
    t = pl.program_id(0)
    rows = lax.broadcasted_iota(jnp.int32, (tm, N), 0) + t * tm
    cols = lax.broadcasted_iota(jnp.int32, (tm, N), 1)
    diag = (rows == cols).astype(jnp.float32)
    for f in range(F):
        acc = filt_ref[f, 0] * a_ref[0]
        for e in range(1, 4):
            acc = acc + filt_ref[f, e] * a_ref[e]
        acc = acc + filt_ref[f, 4] * diag
        out_ref[f] = acc.astype(jnp.bfloat16)


def _mix(filt, A4, tm=256):
    _, N, _ = A4.shape
    tm = min(tm, N)
    F = filt.shape[0]
    return pl.pallas_call(
        _mix_kernel,
        out_shape=jax.ShapeDtypeStruct((F, N, N), jnp.bfloat16),
        grid=(N // tm,),
        in_specs=[
            pl.BlockSpec(memory_space=pltpu.MemorySpace.SMEM),
            pl.BlockSpec((4, tm, N), lambda t: (0, t, 0)),
        ],
        out_specs=pl.BlockSpec((F, tm, N), lambda t: (0, t, 0)),
        compiler_params=pltpu.CompilerParams(dimension_semantics=("parallel",)),
    )(filt, A4)


def _mmnorm_kernel(a_ref, b_ref, o_ref):
    m = jnp.dot(a_ref[0], b_ref[0], preferred_element_type=jnp.float32)
    deg = jnp.sum(m, axis=0, keepdims=True)
    inv = jnp.where(deg == 0.0, 0.0, 1.0 / deg)
    o_ref[0] = (m * inv).astype(jnp.bfloat16)


def _mmnorm(A, B, tn=512):
    C, N, _ = A.shape
    tn = min(tn, N)
    return pl.pallas_call(
        _mmnorm_kernel,
        out_shape=jax.ShapeDtypeStruct((C, N, N), jnp.bfloat16),
        grid=(C, N // tn),
        in_specs=[
            pl.BlockSpec((1, N, N), lambda c, j: (c, 0, 0)),
            pl.BlockSpec((1, N, tn), lambda c, j: (c, 0, j)),
        ],
        out_specs=pl.BlockSpec((1, N, tn), lambda c, j: (c, 0, j)),
        compiler_params=pltpu.CompilerParams(
            dimension_semantics=("parallel", "arbitrary")),
    )(A, B)


def _gcn_mlp_kernel(hn_ref, feat_ref, gw_ref, gb_ref,
                    w1_ref, b1_ref, w2_ref, b2_ref, y_ref, h1_acc):
    c = pl.program_id(0)

    @pl.when(c == 0)
    def _init():
        h1_acc[...] = jnp.zeros_like(h1_acc)

    m = hn_ref[0]
    nz = (m != 0).astype(jnp.float32)
    out_deg = jnp.maximum(jnp.sum(nz, axis=1, keepdims=True), 1.0)
    in_deg = jnp.maximum(jnp.sum(nz, axis=0, keepdims=True), 1.0)

    feat = (feat_ref[...] * lax.rsqrt(out_deg)).astype(jnp.bfloat16)
    agg = lax.dot_general(m, feat,
                          dimension_numbers=(((0,), (0,)), ((), ())),
                          preferred_element_type=jnp.float32)
    agg = agg * lax.rsqrt(in_deg).T
    x_c = jnp.maximum(
        jnp.dot(agg.astype(jnp.bfloat16), gw_ref[...],
                preferred_element_type=jnp.float32) + gb_ref[...],
        0.0)

    h1_acc[...] += jnp.dot(x_c.astype(jnp.bfloat16), w1_ref[...],
                           preferred_element_type=jnp.float32)

    @pl.when(c == pl.num_programs(0) - 1)
    def _finalize():
        h1 = jnp.maximum(h1_acc[...] + b1_ref[...], 0.0)
        y_ref[...] = jnp.dot(h1.astype(jnp.bfloat16), w2_ref[...],
                             preferred_element_type=jnp.float32) + b2_ref[...]


def _gcn_mlp(Hn, feat, gcn_w, gcn_b, lin1_w, lin1_b, lin2_w, lin2_b):
    C, N, _ = Hn.shape
    w_in, w_out = gcn_w.shape
    lin1_out = lin1_w.shape[1]
    num_class = lin2_w.shape[1]
    return pl.pallas_call(
        _gcn_mlp_kernel,
        out_shape=jax.ShapeDtypeStruct((N, num_class), jnp.float32),
        grid=(C,),
        in_specs=[
            pl.BlockSpec((1, N, N), lambda c: (c, 0, 0)),
            pl.BlockSpec((N, w_in), lambda c: (0, 0)),
            pl.BlockSpec((w_in, w_out), lambda c: (0, 0)),
            pl.BlockSpec((1, w_out), lambda c: (0, 0)),
            pl.BlockSpec((w_out, lin1_out), lambda c: (c, 0)),
            pl.BlockSpec((1, lin1_out), lambda c: (0, 0)),
            pl.BlockSpec((lin1_out, num_class), lambda c: (0, 0)),
            pl.BlockSpec((1, num_class), lambda c: (0, 0)),
        ],
        out_specs=pl.BlockSpec((N, num_class), lambda c: (0, 0)),
        scratch_shapes=[pltpu.VMEM((N, lin1_out), jnp.float32)],
        compiler_params=pltpu.CompilerParams(dimension_semantics=("arbitrary",)),
    )(Hn, feat, gcn_w.astype(jnp.bfloat16), gcn_b,
      lin1_w.astype(jnp.bfloat16), lin1_b, lin2_w.astype(jnp.bfloat16), lin2_b)


def kernel(A, h, gt0_w1, gt0_w2, gt1_w1,
           gcn_w, gcn_b, lin1_w, lin1_b, lin2_w, lin2_b):
    C = gt0_w1.shape[0]
    f1 = jax.nn.softmax(gt0_w1, axis=1)
    f2 = jax.nn.softmax(gt0_w2, axis=1)
    fl1 = jax.nn.softmax(gt1_w1, axis=1)
    filt = jnp.concatenate([f1, f2, fl1], axis=0)

    M6 = _mix(filt, A[:4])

    H = _mmnorm(M6[:C], M6[C:2 * C])
    H = _mmnorm(H, M6[2 * C:])

    return _gcn_mlp(H, h, gcn_w, gcn_b, lin1_w, lin1_b, lin2_w, lin2_b)


# speedup vs baseline: 2.8809x; 1.4331x over previous
import jax
import jax.numpy as jnp
from jax import lax
from jax.experimental import pallas as pl
from jax.experimental.pallas import tpu as pltpu


def _mix_kernel(filt_ref, a0_ref, a1_ref, a2_ref, a3_ref, out_ref):
    a = (a0_ref, a1_ref, a2_ref, a3_ref)
    F = out_ref.shape[0]
    tm, N = out_ref.shape[1], out_ref.shape[2]
    t = pl.program_id(0)
    rows = lax.broadcasted_iota(jnp.int32, (tm, N), 0) + t * tm
    cols = lax.broadcasted_iota(jnp.int32, (tm, N), 1)
    diag = (rows == cols).astype(jnp.float32)
    for f in range(F):
        acc = filt_ref[f, 0] * a[0][0]
        for e in range(1, 4):
            acc = acc + filt_ref[f, e] * a[e][0]
        acc = acc + filt_ref[f, 4] * diag
        out_ref[f] = acc.astype(jnp.bfloat16)


def _mix(filt, A, tm=256):
    _, N, _ = A.shape
    tm = min(tm, N)
    F = filt.shape[0]

    def chan(e):
        return pl.BlockSpec((1, tm, N), lambda t, e=e: (e, t, 0))

    return pl.pallas_call(
        _mix_kernel,
        out_shape=jax.ShapeDtypeStruct((F, N, N), jnp.bfloat16),
        grid=(N // tm,),
        in_specs=[
            pl.BlockSpec(memory_space=pltpu.MemorySpace.SMEM),
            chan(0), chan(1), chan(2), chan(3),
        ],
        out_specs=pl.BlockSpec((F, tm, N), lambda t: (0, t, 0)),
        compiler_params=pltpu.CompilerParams(dimension_semantics=("parallel",)),
    )(filt, A, A, A, A)


def _mmnorm_kernel(a_ref, b_ref, o_ref):
    m = jnp.dot(a_ref[0], b_ref[0], preferred_element_type=jnp.float32)
    deg = jnp.sum(m, axis=0, keepdims=True)
    inv = jnp.where(deg == 0.0, 0.0, 1.0 / deg)
    o_ref[0] = (m * inv).astype(jnp.bfloat16)


def _mmnorm(A, B, C, a_off=0, b_off=0, tn=512):
    N = A.shape[1]
    tn = min(tn, N)
    return pl.pallas_call(
        _mmnorm_kernel,
        out_shape=jax.ShapeDtypeStruct((C, N, N), jnp.bfloat16),
        grid=(C, N // tn),
        in_specs=[
            pl.BlockSpec((1, N, N), lambda c, j: (c + a_off, 0, 0)),
            pl.BlockSpec((1, N, tn), lambda c, j: (c + b_off, 0, j)),
        ],
        out_specs=pl.BlockSpec((1, N, tn), lambda c, j: (c, 0, j)),
        compiler_params=pltpu.CompilerParams(
            dimension_semantics=("parallel", "arbitrary")),
    )(A, B)


def _gcn_mlp_kernel(hn_ref, feat_ref, gw_ref, gb_ref,
                    w1_ref, b1_ref, w2_ref, b2_ref, y_ref, h1_acc):
    c = pl.program_id(0)

    @pl.when(c == 0)
    def _init():
        h1_acc[...] = jnp.zeros_like(h1_acc)

    m = hn_ref[0]
    nz = (m != 0).astype(jnp.float32)
    out_deg = jnp.maximum(jnp.sum(nz, axis=1, keepdims=True), 1.0)
    in_deg = jnp.maximum(jnp.sum(nz, axis=0, keepdims=True), 1.0)

    feat = (feat_ref[...] * lax.rsqrt(out_deg)).astype(jnp.bfloat16)
    agg = lax.dot_general(m, feat,
                          dimension_numbers=(((0,), (0,)), ((), ())),
                          preferred_element_type=jnp.float32)
    agg = agg * lax.rsqrt(in_deg).T
    x_c = jnp.maximum(
        jnp.dot(agg.astype(jnp.bfloat16), gw_ref[...],
                preferred_element_type=jnp.float32) + gb_ref[...],
        0.0)

    h1_acc[...] += jnp.dot(x_c.astype(jnp.bfloat16), w1_ref[...],
                           preferred_element_type=jnp.float32)

    @pl.when(c == pl.num_programs(0) - 1)
    def _finalize():
        h1 = jnp.maximum(h1_acc[...] + b1_ref[...], 0.0)
        y_ref[...] = jnp.dot(h1.astype(jnp.bfloat16), w2_ref[...],
                             preferred_element_type=jnp.float32) + b2_ref[...]


def _gcn_mlp(Hn, feat, gcn_w, gcn_b, lin1_w, lin1_b, lin2_w, lin2_b):
    C, N, _ = Hn.shape
    w_in, w_out = gcn_w.shape
    lin1_out = lin1_w.shape[1]
    num_class = lin2_w.shape[1]
    return pl.pallas_call(
        _gcn_mlp_kernel,
        out_shape=jax.ShapeDtypeStruct((N, num_class), jnp.float32),
        grid=(C,),
        in_specs=[
            pl.BlockSpec((1, N, N), lambda c: (c, 0, 0)),
            pl.BlockSpec((N, w_in), lambda c: (0, 0)),
            pl.BlockSpec((w_in, w_out), lambda c: (0, 0)),
            pl.BlockSpec((1, w_out), lambda c: (0, 0)),
            pl.BlockSpec((w_out, lin1_out), lambda c: (c, 0)),
            pl.BlockSpec((1, lin1_out), lambda c: (0, 0)),
            pl.BlockSpec((lin1_out, num_class), lambda c: (0, 0)),
            pl.BlockSpec((1, num_class), lambda c: (0, 0)),
        ],
        out_specs=pl.BlockSpec((N, num_class), lambda c: (0, 0)),
        scratch_shapes=[pltpu.VMEM((N, lin1_out), jnp.float32)],
        compiler_params=pltpu.CompilerParams(dimension_semantics=("arbitrary",)),
    )(Hn, feat, gcn_w.astype(jnp.bfloat16), gcn_b,
      lin1_w.astype(jnp.bfloat16), lin1_b, lin2_w.astype(jnp.bfloat16), lin2_b)


def kernel(A, h, gt0_w1, gt0_w2, gt1_w1,
           gcn_w, gcn_b, lin1_w, lin1_b, lin2_w, lin2_b):
    C = gt0_w1.shape[0]
    f1 = jax.nn.softmax(gt0_w1, axis=1)
    f2 = jax.nn.softmax(gt0_w2, axis=1)
    fl1 = jax.nn.softmax(gt1_w1, axis=1)
    filt = jnp.concatenate([f1, f2, fl1], axis=0)

    M6 = _mix(filt, A)

    H = _mmnorm(M6, M6, C, a_off=0, b_off=C)
    H = _mmnorm(H, M6, C, a_off=0, b_off=2 * C)

    return _gcn_mlp(H, h, gcn_w, gcn_b, lin1_w, lin1_b, lin2_w, lin2_b)


# speedup vs baseline: 2.9463x; 1.0227x over previous
import jax
import jax.numpy as jnp
from jax import lax
from jax.experimental import pallas as pl
from jax.experimental.pallas import tpu as pltpu


def _mix_kernel(filt_ref, a0_ref, a1_ref, a2_ref, a3_ref, out_ref):
    a = (a0_ref, a1_ref, a2_ref, a3_ref)
    F = out_ref.shape[0]
    tm, N = out_ref.shape[1], out_ref.shape[2]
    t = pl.program_id(0)
    rows = lax.broadcasted_iota(jnp.int32, (tm, N), 0) + t * tm
    cols = lax.broadcasted_iota(jnp.int32, (tm, N), 1)
    diag = (rows == cols).astype(jnp.float32)
    for f in range(F):
        acc = filt_ref[f, 0] * a[0][0]
        for e in range(1, 4):
            acc = acc + filt_ref[f, e] * a[e][0]
        acc = acc + filt_ref[f, 4] * diag
        out_ref[f] = acc.astype(jnp.bfloat16)


def _mix(filt, A, tm=256):
    _, N, _ = A.shape
    tm = min(tm, N)
    F = filt.shape[0]

    def chan(e):
        return pl.BlockSpec((1, tm, N), lambda t, e=e: (e, t, 0))

    return pl.pallas_call(
        _mix_kernel,
        out_shape=jax.ShapeDtypeStruct((F, N, N), jnp.bfloat16),
        grid=(N // tm,),
        in_specs=[
            pl.BlockSpec(memory_space=pltpu.MemorySpace.SMEM),
            chan(0), chan(1), chan(2), chan(3),
        ],
        out_specs=pl.BlockSpec((F, tm, N), lambda t: (0, t, 0)),
        compiler_params=pltpu.CompilerParams(dimension_semantics=("parallel",)),
    )(filt, A, A, A, A)


def _mega_kernel(a_ref, b_ref, feat_ref, gw_ref, gb_ref, w1_ref, o_ref,
                 h0, h1, ideg, odeg, featb):
    p = pl.program_id(1)
    j = pl.program_id(2)
    tn = b_ref.shape[2]

    def colnorm(m):
        deg = jnp.sum(m, axis=0, keepdims=True)
        inv = jnp.where(deg == 0.0, 0.0, 1.0 / deg)
        return (m * inv).astype(jnp.bfloat16)

    @pl.when(p == 0)
    def _layer0():
        m = jnp.dot(a_ref[0], b_ref[0], preferred_element_type=jnp.float32)
        h0[:, pl.ds(j * tn, tn)] = colnorm(m)

    @pl.when(p == 1)
    def _layer1():
        m = jnp.dot(h0[...], b_ref[0], preferred_element_type=jnp.float32)
        h1[:, pl.ds(j * tn, tn)] = colnorm(m)
        nz = (m != 0.0).astype(jnp.float32)
        ideg[:, pl.ds(j * tn, tn)] = jnp.sum(nz, axis=0, keepdims=True)
        rc = jnp.sum(nz, axis=1, keepdims=True)

        @pl.when(j == 0)
        def _():
            odeg[...] = rc

        @pl.when(j > 0)
        def _():
            odeg[...] += rc

    @pl.when(p == 2)
    def _gcn():
        @pl.when(j == 0)
        def _():
            od = jnp.maximum(odeg[...], 1.0)
            featb[...] = (feat_ref[...] * lax.rsqrt(od)).astype(jnp.bfloat16)

        mblk = h1[:, pl.ds(j * tn, tn)]
        agg = lax.dot_general(mblk, featb[...],
                              dimension_numbers=(((0,), (0,)), ((), ())),
                              preferred_element_type=jnp.float32)
        idg = jnp.maximum(ideg[:, pl.ds(j * tn, tn)], 1.0)
        agg = agg * lax.rsqrt(idg).T
        x = jnp.maximum(
            jnp.dot(agg.astype(jnp.bfloat16), gw_ref[...],
                    preferred_element_type=jnp.float32) + gb_ref[...], 0.0)
        o_ref[0] = jnp.dot(x.astype(jnp.bfloat16), w1_ref[...],
                           preferred_element_type=jnp.float32)


def _mega(M6, C, feat, gcn_w, gcn_b, lin1_w, tn=512):
    N = M6.shape[1]
    tn = min(tn, N)
    J = N // tn
    w_in = feat.shape[1]
    w_out = gcn_w.shape[1]
    return pl.pallas_call(
        _mega_kernel,
        out_shape=jax.ShapeDtypeStruct((C, N, w_out), jnp.float32),
        grid=(C, 3, J),
        in_specs=[
            pl.BlockSpec((1, N, N), lambda c, p, j: (c, 0, 0)),
            pl.BlockSpec((1, N, tn),
                         lambda c, p, j: (jnp.minimum(C + p * C + c,
                                                      3 * C - 1), 0, j)),
            pl.BlockSpec((N, w_in), lambda c, p, j: (0, 0)),
            pl.BlockSpec((w_in, w_out), lambda c, p, j: (0, 0)),
            pl.BlockSpec((1, w_out), lambda c, p, j: (0, 0)),
            pl.BlockSpec((w_out, w_out), lambda c, p, j: (c, 0)),
        ],
        out_specs=pl.BlockSpec(
            (1, tn, w_out),
            lambda c, p, j: (c, jnp.where(p == 2, j, 0), 0)),
        scratch_shapes=[
            pltpu.VMEM((N, N), jnp.bfloat16),
            pltpu.VMEM((N, N), jnp.bfloat16),
            pltpu.VMEM((1, N), jnp.float32),
            pltpu.VMEM((N, 1), jnp.float32),
            pltpu.VMEM((N, 128), jnp.bfloat16),
        ],
        compiler_params=pltpu.CompilerParams(
            dimension_semantics=("parallel", "arbitrary", "arbitrary")),
    )(M6, M6, feat, gcn_w.astype(jnp.bfloat16), gcn_b,
      lin1_w.astype(jnp.bfloat16))


def _combine_kernel(p_ref, b1_ref, w2_ref, b2_ref, y_ref):
    h1 = p_ref[0]
    for c in range(1, p_ref.shape[0]):
        h1 = h1 + p_ref[c]
    h1 = jnp.maximum(h1 + b1_ref[...], 0.0)
    y_ref[...] = jnp.dot(h1.astype(jnp.bfloat16), w2_ref[...],
                         preferred_element_type=jnp.float32) + b2_ref[...]


def _combine(parts, lin1_b, lin2_w, lin2_b):
    C, N, w_out = parts.shape
    num_class = lin2_w.shape[1]
    return pl.pallas_call(
        _combine_kernel,
        out_shape=jax.ShapeDtypeStruct((N, num_class), jnp.float32),
        in_specs=[
            pl.BlockSpec((C, N, w_out), lambda: (0, 0, 0)),
            pl.BlockSpec((1, w_out), lambda: (0, 0)),
            pl.BlockSpec((w_out, num_class), lambda: (0, 0)),
            pl.BlockSpec((1, num_class), lambda: (0, 0)),
        ],
        out_specs=pl.BlockSpec((N, num_class), lambda: (0, 0)),
    )(parts, lin1_b, lin2_w.astype(jnp.bfloat16), lin2_b)


def kernel(A, h, gt0_w1, gt0_w2, gt1_w1,
           gcn_w, gcn_b, lin1_w, lin1_b, lin2_w, lin2_b):
    C = gt0_w1.shape[0]
    f1 = jax.nn.softmax(gt0_w1, axis=1)
    f2 = jax.nn.softmax(gt0_w2, axis=1)
    fl1 = jax.nn.softmax(gt1_w1, axis=1)
    filt = jnp.concatenate([f1, f2, fl1], axis=0)

    M6 = _mix(filt, A)

    parts = _mega(M6, C, h, gcn_w, gcn_b, lin1_w)
    return _combine(parts, lin1_b, lin2_w, lin2_b)


# speedup vs baseline: 3.0798x; 1.0453x over previous
import jax
import jax.numpy as jnp
from jax import lax
from jax.experimental import pallas as pl
from jax.experimental.pallas import tpu as pltpu


def _mix_kernel(filt_ref, a0_ref, a1_ref, a2_ref, a3_ref, out_ref):
    a = (a0_ref, a1_ref, a2_ref, a3_ref)
    F = out_ref.shape[0]
    tm, N = out_ref.shape[1], out_ref.shape[2]
    t = pl.program_id(0)
    rows = lax.broadcasted_iota(jnp.int32, (tm, N), 0) + t * tm
    cols = lax.broadcasted_iota(jnp.int32, (tm, N), 1)
    diag = (rows == cols).astype(jnp.float32)
    for f in range(F):
        acc = filt_ref[f, 0] * a[0][0]
        for e in range(1, 4):
            acc = acc + filt_ref[f, e] * a[e][0]
        acc = acc + filt_ref[f, 4] * diag
        out_ref[f] = acc.astype(jnp.bfloat16)


def _mix(filt, A, tm=256):
    _, N, _ = A.shape
    tm = min(tm, N)
    F = filt.shape[0]

    def chan(e):
        return pl.BlockSpec((1, tm, N), lambda t, e=e: (e, t, 0))

    return pl.pallas_call(
        _mix_kernel,
        out_shape=jax.ShapeDtypeStruct((F, N, N), jnp.bfloat16),
        grid=(N // tm,),
        in_specs=[
            pl.BlockSpec(memory_space=pltpu.MemorySpace.SMEM),
            chan(0), chan(1), chan(2), chan(3),
        ],
        out_specs=pl.BlockSpec((F, tm, N), lambda t: (0, t, 0)),
        compiler_params=pltpu.CompilerParams(dimension_semantics=("parallel",)),
    )(filt, A, A, A, A)


def _mega_kernel(a_ref, b_ref, feat_ref, gw_ref, gb_ref, w1_ref, o_ref,
                 h0, h1, ideg, odeg, featb):
    p = pl.program_id(1)
    j = pl.program_id(2)
    tn = b_ref.shape[2]

    def colnorm(m):
        deg = jnp.sum(m, axis=0, keepdims=True)
        inv = jnp.where(deg == 0.0, 0.0, 1.0 / deg)
        return (m * inv).astype(jnp.bfloat16)

    @pl.when(p == 0)
    def _layer0():
        m = jnp.dot(a_ref[0], b_ref[0], preferred_element_type=jnp.float32)
        h0[:, pl.ds(j * tn, tn)] = colnorm(m)

    @pl.when(p == 1)
    def _layer1():
        m = jnp.dot(h0[...], b_ref[0], preferred_element_type=jnp.float32)
        h1[:, pl.ds(j * tn, tn)] = colnorm(m)
        nz = (m != 0.0).astype(jnp.float32)
        ideg[:, pl.ds(j * tn, tn)] = jnp.sum(nz, axis=0, keepdims=True)
        rc = jnp.sum(nz, axis=1, keepdims=True)

        @pl.when(j == 0)
        def _():
            odeg[...] = rc

        @pl.when(j > 0)
        def _():
            odeg[...] += rc

    @pl.when(p == 2)
    def _gcn():
        @pl.when(j == 0)
        def _():
            od = jnp.maximum(odeg[...], 1.0)
            featb[...] = (feat_ref[...] * lax.rsqrt(od)).astype(jnp.bfloat16)

        mblk = h1[:, pl.ds(j * tn, tn)]
        agg = lax.dot_general(mblk, featb[...],
                              dimension_numbers=(((0,), (0,)), ((), ())),
                              preferred_element_type=jnp.float32)
        idg = jnp.maximum(ideg[:, pl.ds(j * tn, tn)], 1.0)
        agg = agg * lax.rsqrt(idg).T
        x = jnp.maximum(
            jnp.dot(agg.astype(jnp.bfloat16), gw_ref[...],
                    preferred_element_type=jnp.float32) + gb_ref[...], 0.0)
        o_ref[0] = jnp.dot(x.astype(jnp.bfloat16), w1_ref[...],
                           preferred_element_type=jnp.float32)


def _mega(M6, C, feat, gcn_w, gcn_b, lin1_w, tn=1024):
    N = M6.shape[1]
    tn = min(tn, N)
    J = N // tn
    w_in = feat.shape[1]
    w_out = gcn_w.shape[1]
    return pl.pallas_call(
        _mega_kernel,
        out_shape=jax.ShapeDtypeStruct((C, N, w_out), jnp.float32),
        grid=(C, 3, J),
        in_specs=[
            pl.BlockSpec((1, N, N), lambda c, p, j: (c, 0, 0)),
            pl.BlockSpec((1, N, tn),
                         lambda c, p, j: (C + jnp.minimum(p, 1) * C + c, 0,
                                          jnp.where(p == 2, J - 1, j))),
            pl.BlockSpec((N, w_in), lambda c, p, j: (0, 0)),
            pl.BlockSpec((w_in, w_out), lambda c, p, j: (0, 0)),
            pl.BlockSpec((1, w_out), lambda c, p, j: (0, 0)),
            pl.BlockSpec((w_out, w_out), lambda c, p, j: (c, 0)),
        ],
        out_specs=pl.BlockSpec(
            (1, tn, w_out),
            lambda c, p, j: (c, jnp.where(p == 2, j, 0), 0)),
        scratch_shapes=[
            pltpu.VMEM((N, N), jnp.bfloat16),
            pltpu.VMEM((N, N), jnp.bfloat16),
            pltpu.VMEM((1, N), jnp.float32),
            pltpu.VMEM((N, 1), jnp.float32),
            pltpu.VMEM((N, 128), jnp.bfloat16),
        ],
        compiler_params=pltpu.CompilerParams(
            dimension_semantics=("parallel", "arbitrary", "arbitrary")),
    )(M6, M6, feat, gcn_w.astype(jnp.bfloat16), gcn_b,
      lin1_w.astype(jnp.bfloat16))


def _combine_kernel(p_ref, b1_ref, w2_ref, b2_ref, y_ref):
    h1 = p_ref[0]
    for c in range(1, p_ref.shape[0]):
        h1 = h1 + p_ref[c]
    h1 = jnp.maximum(h1 + b1_ref[...], 0.0)
    y_ref[...] = jnp.dot(h1.astype(jnp.bfloat16), w2_ref[...],
                         preferred_element_type=jnp.float32) + b2_ref[...]


def _combine(parts, lin1_b, lin2_w, lin2_b):
    C, N, w_out = parts.shape
    num_class = lin2_w.shape[1]
    return pl.pallas_call(
        _combine_kernel,
        out_shape=jax.ShapeDtypeStruct((N, num_class), jnp.float32),
        in_specs=[
            pl.BlockSpec((C, N, w_out), lambda: (0, 0, 0)),
            pl.BlockSpec((1, w_out), lambda: (0, 0)),
            pl.BlockSpec((w_out, num_class), lambda: (0, 0)),
            pl.BlockSpec((1, num_class), lambda: (0, 0)),
        ],
        out_specs=pl.BlockSpec((N, num_class), lambda: (0, 0)),
    )(parts, lin1_b, lin2_w.astype(jnp.bfloat16), lin2_b)


def kernel(A, h, gt0_w1, gt0_w2, gt1_w1,
           gcn_w, gcn_b, lin1_w, lin1_b, lin2_w, lin2_b):
    C = gt0_w1.shape[0]
    f1 = jax.nn.softmax(gt0_w1, axis=1)
    f2 = jax.nn.softmax(gt0_w2, axis=1)
    fl1 = jax.nn.softmax(gt1_w1, axis=1)
    filt = jnp.concatenate([f1, f2, fl1], axis=0)

    M6 = _mix(filt, A)

    parts = _mega(M6, C, h, gcn_w, gcn_b, lin1_w)
    return _combine(parts, lin1_b, lin2_w, lin2_b)
